# baseline (speedup 1.0000x reference)
import functools

import jax
import jax.numpy as jnp
from jax import lax
from jax.experimental import pallas as pl
from jax.experimental.pallas import tpu as pltpu
from jax.experimental.pallas import tpu_sc as plsc

D_MODEL = 768
NUM_EXPERTS = 16
D_FF = 4 * D_MODEL
N_TOK = 2048
TILE = 128
NT = N_TOK // TILE + NUM_EXPERTS
P = NT * TILE
NC, NS = 2, 16
NW = NC * NS
LANES = 16


def _route_body(x_ref, gw_ref, gb_ref, pos_ref, wtok_ref, ex_ref, valid_ref):
    x = x_ref[...]
    gw = gw_ref[...]
    gb = gb_ref[...]
    logits = jnp.dot(x, gw, preferred_element_type=jnp.float32) + gb
    m = jnp.max(logits, axis=-1, keepdims=True)
    e = jnp.exp(logits - m)
    w = e / jnp.sum(e, axis=-1, keepdims=True)
    wmax = jnp.max(w, axis=-1, keepdims=True)
    eids = lax.broadcasted_iota(jnp.int32, (N_TOK, NUM_EXPERTS), 1)
    expert = jnp.min(jnp.where(w == wmax, eids, NUM_EXPERTS), axis=-1,
                     keepdims=True)
    onehot = (eids == expert).astype(jnp.float32)

    stl_r = lax.broadcasted_iota(jnp.int32, (TILE, TILE), 0)
    stl_c = lax.broadcasted_iota(jnp.int32, (TILE, TILE), 1)
    stl = (stl_c < stl_r).astype(jnp.float32)
    ranks = []
    counts = jnp.zeros((1, NUM_EXPERTS), jnp.float32)
    for c in range(N_TOK // TILE):
        oh_c = onehot[c * TILE:(c + 1) * TILE, :]
        cs = jnp.dot(stl, oh_c, preferred_element_type=jnp.float32) + counts
        ranks.append(jnp.sum(cs * oh_c, axis=-1, keepdims=True))
        counts = counts + jnp.sum(oh_c, axis=0, keepdims=True)
    rank = jnp.concatenate(ranks, axis=0)

    tiles = jnp.ceil(counts / TILE)
    ee_r = lax.broadcasted_iota(jnp.int32, (NUM_EXPERTS, NUM_EXPERTS), 0)
    ee_c = lax.broadcasted_iota(jnp.int32, (NUM_EXPERTS, NUM_EXPERTS), 1)
    excl = jnp.dot(tiles, (ee_r < ee_c).astype(jnp.float32),
                   preferred_element_type=jnp.float32)
    inc = excl + tiles
    pad_start = TILE * excl

    pos = jnp.sum(onehot * pad_start, axis=-1, keepdims=True) + rank
    pos_ref[...] = pos.astype(jnp.int32)
    wtok_ref[...] = wmax

    total = jnp.max(inc)
    it = lax.broadcasted_iota(jnp.int32, (NT, NUM_EXPERTS), 0).astype(jnp.float32)
    ex_full = jnp.sum((jnp.broadcast_to(inc, (NT, NUM_EXPERTS)) <= it)
                      .astype(jnp.int32), axis=-1, keepdims=True)
    ex_last = jnp.sum((inc <= total - 1.0).astype(jnp.int32), axis=-1,
                      keepdims=True)
    tcol = lax.broadcasted_iota(jnp.int32, (NT, 1), 0).astype(jnp.float32)
    valid = tcol < total
    ex = jnp.where(valid, jnp.minimum(ex_full, NUM_EXPERTS - 1),
                   jnp.broadcast_to(ex_last, (NT, 1)))
    ex_ref[...] = ex
    valid_ref[...] = valid.astype(jnp.int32)


def _route(x_flat, gate_w, gate_b):
    return pl.pallas_call(
        _route_body,
        out_shape=[
            jax.ShapeDtypeStruct((N_TOK, 1), jnp.int32),
            jax.ShapeDtypeStruct((N_TOK, 1), jnp.float32),
            jax.ShapeDtypeStruct((NT, 1), jnp.int32),
            jax.ShapeDtypeStruct((NT, 1), jnp.int32),
        ],
    )(x_flat, gate_w, gate_b.reshape(1, NUM_EXPERTS))


@functools.lru_cache(maxsize=None)
def _build_sc_scatter():
    mesh = plsc.VectorSubcoreMesh(core_axis_name="c", subcore_axis_name="s",
                                  num_cores=NC, num_subcores=NS)

    @functools.partial(
        pl.kernel,
        out_type=[
            jax.ShapeDtypeStruct((P, D_MODEL), jnp.float32),
            jax.ShapeDtypeStruct((P,), jnp.float32),
        ],
        mesh=mesh,
        scratch_types=[
            pltpu.VMEM((_TOK_W,), jnp.int32),
            pltpu.VMEM((_TOK_W,), jnp.float32),
            pltpu.VMEM((_TOK_W, D_MODEL), jnp.float32),
            pltpu.SemaphoreType.DMA,
        ],
        compiler_params=pltpu.CompilerParams(needs_layout_passes=False),
    )
    def _sc_scatter(x_hbm, pos_hbm, wtok_hbm, xs_hbm, ws_hbm,
                    pos_v, wtok_v, rows_v, sem):
        wid = lax.axis_index("s") * NC + lax.axis_index("c")
        base = wid * _TOK_W
        pltpu.sync_copy(pos_hbm.at[pl.ds(base, _TOK_W)], pos_v)
        pltpu.sync_copy(x_hbm.at[pl.ds(base, _TOK_W)], rows_v)
        pltpu.async_copy(rows_v, xs_hbm.at[pos_v], sem).wait()
        pltpu.sync_copy(wtok_hbm.at[pl.ds(base, _TOK_W)], wtok_v)
        pltpu.async_copy(wtok_v, ws_hbm.at[pos_v], sem).wait()

    return _sc_scatter


def _ffn_body(ex_ref, valid_ref, xs_ref, w1_ref, b1_ref, w2_ref, b2_ref,
              ws_ref, ys_ref):
    i = pl.program_id(0)

    @pl.when(valid_ref[i] != 0)
    def _():
        xt = xs_ref[...].astype(jnp.bfloat16)
        h = jnp.dot(xt, w1_ref[0], preferred_element_type=jnp.float32)
        h = h + b1_ref[0]
        h = h * 0.5 * (1.0 + lax.erf(h * 0.7071067811865476))
        y = jnp.dot(h.astype(jnp.bfloat16), w2_ref[0],
                    preferred_element_type=jnp.float32)
        ys_ref[...] = (y + b2_ref[0]) * ws_ref[...]


def _ffn(ex, valid, xs, w1, b1, w2, b2, ws_col):
    grid_spec = pltpu.PrefetchScalarGridSpec(
        num_scalar_prefetch=2,
        grid=(NT,),
        in_specs=[
            pl.BlockSpec((TILE, D_MODEL), lambda i, ex, v: (i, 0)),
            pl.BlockSpec((1, D_MODEL, D_FF), lambda i, ex, v: (ex[i], 0, 0)),
            pl.BlockSpec((1, 1, D_FF), lambda i, ex, v: (ex[i], 0, 0)),
            pl.BlockSpec((1, D_FF, D_MODEL), lambda i, ex, v: (ex[i], 0, 0)),
            pl.BlockSpec((1, 1, D_MODEL), lambda i, ex, v: (ex[i], 0, 0)),
            pl.BlockSpec((TILE, 1), lambda i, ex, v: (i, 0)),
        ],
        out_specs=pl.BlockSpec((TILE, D_MODEL), lambda i, ex, v: (i, 0)),
    )
    return pl.pallas_call(
        _ffn_body,
        grid_spec=grid_spec,
        out_shape=jax.ShapeDtypeStruct((P, D_MODEL), jnp.float32),
        compiler_params=pltpu.CompilerParams(
            dimension_semantics=("arbitrary",),
            vmem_limit_bytes=100 * 1024 * 1024),
    )(ex, valid, xs,
      w1.astype(jnp.bfloat16), b1.reshape(NUM_EXPERTS, 1, D_FF),
      w2.astype(jnp.bfloat16), b2.reshape(NUM_EXPERTS, 1, D_MODEL), ws_col)


_TOK_W = N_TOK // NW


@functools.lru_cache(maxsize=None)
def _build_sc_gather():
    mesh = plsc.VectorSubcoreMesh(core_axis_name="c", subcore_axis_name="s",
                                  num_cores=NC, num_subcores=NS)

    @functools.partial(
        pl.kernel,
        out_type=jax.ShapeDtypeStruct((N_TOK, D_MODEL), jnp.float32),
        mesh=mesh,
        scratch_types=[
            pltpu.VMEM((_TOK_W,), jnp.int32),
            pltpu.VMEM((_TOK_W, D_MODEL), jnp.float32),
            pltpu.SemaphoreType.DMA,
        ],
        compiler_params=pltpu.CompilerParams(needs_layout_passes=False),
    )
    def _sc_gather(ys_hbm, pos_hbm, out_hbm, pos_v, rows_v, sem):
        wid = lax.axis_index("s") * NC + lax.axis_index("c")
        pltpu.sync_copy(pos_hbm.at[pl.ds(wid * _TOK_W, _TOK_W)], pos_v)
        pltpu.async_copy(ys_hbm.at[pos_v], rows_v, sem).wait()
        pltpu.sync_copy(rows_v, out_hbm.at[pl.ds(wid * _TOK_W, _TOK_W)])

    return _sc_gather


def kernel(x, gate_w, gate_b, w1, b1, w2, b2):
    batch, seq_len, hidden = x.shape
    x_flat = x.reshape(N_TOK, D_MODEL)
    pos2, wtok2, ex2, valid2 = _route(x_flat, gate_w, gate_b)
    pos = pos2.reshape(N_TOK)
    wtok = wtok2.reshape(N_TOK)
    xs, ws = _build_sc_scatter()(x_flat, pos, wtok)
    ys = _ffn(ex2.reshape(NT), valid2.reshape(NT), xs,
              w1, b1, w2, b2, ws.reshape(P, 1))
    out = _build_sc_gather()(ys, pos)
    return out.reshape(batch, seq_len, hidden)


# speedup vs baseline: 1.3943x; 1.3943x over previous
import functools

import jax
import jax.numpy as jnp
from jax import lax
from jax.experimental import pallas as pl
from jax.experimental.pallas import tpu as pltpu
from jax.experimental.pallas import tpu_sc as plsc

D_MODEL = 768
NUM_EXPERTS = 16
D_FF = 4 * D_MODEL
N_TOK = 2048
TILE = 128
NT = N_TOK // TILE + NUM_EXPERTS
P = NT * TILE
NC, NS = 2, 16
NW = NC * NS
LANES = 16


def _route_body(x_ref, gw_ref, gb_ref, pos_ref, wtok_ref, ex_ref, valid_ref):
    x = x_ref[...]
    gw = gw_ref[...]
    gb = gb_ref[...]
    logits = jnp.dot(x, gw, preferred_element_type=jnp.float32) + gb
    m = jnp.max(logits, axis=-1, keepdims=True)
    e = jnp.exp(logits - m)
    w = e / jnp.sum(e, axis=-1, keepdims=True)
    wmax = jnp.max(w, axis=-1, keepdims=True)
    eids = lax.broadcasted_iota(jnp.int32, (N_TOK, NUM_EXPERTS), 1)
    expert = jnp.min(jnp.where(w == wmax, eids, NUM_EXPERTS), axis=-1,
                     keepdims=True)
    onehot = (eids == expert).astype(jnp.float32)

    stl_r = lax.broadcasted_iota(jnp.int32, (TILE, TILE), 0)
    stl_c = lax.broadcasted_iota(jnp.int32, (TILE, TILE), 1)
    stl = (stl_c < stl_r).astype(jnp.float32)
    ranks = []
    counts = jnp.zeros((1, NUM_EXPERTS), jnp.float32)
    for c in range(N_TOK // TILE):
        oh_c = onehot[c * TILE:(c + 1) * TILE, :]
        cs = jnp.dot(stl, oh_c, preferred_element_type=jnp.float32) + counts
        ranks.append(jnp.sum(cs * oh_c, axis=-1, keepdims=True))
        counts = counts + jnp.sum(oh_c, axis=0, keepdims=True)
    rank = jnp.concatenate(ranks, axis=0)

    tiles = jnp.ceil(counts / TILE)
    ee_r = lax.broadcasted_iota(jnp.int32, (NUM_EXPERTS, NUM_EXPERTS), 0)
    ee_c = lax.broadcasted_iota(jnp.int32, (NUM_EXPERTS, NUM_EXPERTS), 1)
    excl = jnp.dot(tiles, (ee_r < ee_c).astype(jnp.float32),
                   preferred_element_type=jnp.float32)
    inc = excl + tiles
    pad_start = TILE * excl

    pos = jnp.sum(onehot * pad_start, axis=-1, keepdims=True) + rank
    pos_ref[...] = pos.astype(jnp.int32)
    wtok_ref[...] = wmax

    total = jnp.max(inc)
    it = lax.broadcasted_iota(jnp.int32, (NT, NUM_EXPERTS), 0).astype(jnp.float32)
    ex_full = jnp.sum((jnp.broadcast_to(inc, (NT, NUM_EXPERTS)) <= it)
                      .astype(jnp.int32), axis=-1, keepdims=True)
    ex_last = jnp.sum((inc <= total - 1.0).astype(jnp.int32), axis=-1,
                      keepdims=True)
    tcol = lax.broadcasted_iota(jnp.int32, (NT, 1), 0).astype(jnp.float32)
    valid = tcol < total
    ex = jnp.where(valid, jnp.minimum(ex_full, NUM_EXPERTS - 1),
                   jnp.broadcast_to(ex_last, (NT, 1)))
    ex_ref[...] = ex
    valid_ref[...] = valid.astype(jnp.int32)


def _route(x_flat, gate_w, gate_b):
    return pl.pallas_call(
        _route_body,
        out_shape=[
            jax.ShapeDtypeStruct((N_TOK, 1), jnp.int32),
            jax.ShapeDtypeStruct((N_TOK, 1), jnp.float32),
            jax.ShapeDtypeStruct((NT, 1), jnp.int32),
            jax.ShapeDtypeStruct((NT, 1), jnp.int32),
        ],
    )(x_flat, gate_w, gate_b.reshape(1, NUM_EXPERTS))


@functools.lru_cache(maxsize=None)
def _build_sc_scatter():
    mesh = plsc.VectorSubcoreMesh(core_axis_name="c", subcore_axis_name="s",
                                  num_cores=NC, num_subcores=NS)

    @functools.partial(
        pl.kernel,
        out_type=[
            jax.ShapeDtypeStruct((P, D_MODEL), jnp.float32),
            jax.ShapeDtypeStruct((P,), jnp.float32),
        ],
        mesh=mesh,
        scratch_types=[
            pltpu.VMEM((_TOK_W,), jnp.int32),
            pltpu.VMEM((_TOK_W,), jnp.float32),
            pltpu.VMEM((_TOK_W, D_MODEL), jnp.float32),
            pltpu.SemaphoreType.DMA,
            pltpu.SemaphoreType.DMA,
            pltpu.SemaphoreType.DMA,
        ],
        compiler_params=pltpu.CompilerParams(needs_layout_passes=False),
    )
    def _sc_scatter(x_hbm, pos_hbm, wtok_hbm, xs_hbm, ws_hbm,
                    pos_v, wtok_v, rows_v, sem_p, sem_r, sem_w):
        wid = lax.axis_index("s") * NC + lax.axis_index("c")
        base = wid * _TOK_W
        cp_p = pltpu.async_copy(pos_hbm.at[pl.ds(base, _TOK_W)], pos_v, sem_p)
        cp_r = pltpu.async_copy(x_hbm.at[pl.ds(base, _TOK_W)], rows_v, sem_r)
        cp_w = pltpu.async_copy(wtok_hbm.at[pl.ds(base, _TOK_W)], wtok_v, sem_w)
        cp_p.wait()
        cp_w.wait()
        s_w = pltpu.async_copy(wtok_v, ws_hbm.at[pos_v], sem_w)
        cp_r.wait()
        s_r = pltpu.async_copy(rows_v, xs_hbm.at[pos_v], sem_r)
        s_w.wait()
        s_r.wait()

    return _sc_scatter


def _ffn_body(ex_ref, valid_ref, xs_ref, w1_ref, b1_ref, w2_ref, b2_ref,
              ws_ref, ys_ref):
    i = pl.program_id(0)

    @pl.when(valid_ref[i] != 0)
    def _():
        xt = xs_ref[...].astype(jnp.bfloat16)
        h = jnp.dot(xt, w1_ref[0].astype(jnp.bfloat16),
                    preferred_element_type=jnp.float32)
        h = h + b1_ref[0]
        h = h * 0.5 * (1.0 + lax.erf(h * 0.7071067811865476))
        y = jnp.dot(h.astype(jnp.bfloat16), w2_ref[0].astype(jnp.bfloat16),
                    preferred_element_type=jnp.float32)
        ys_ref[...] = (y + b2_ref[0]) * ws_ref[...]


def _ffn(ex, valid, xs, w1, b1, w2, b2, ws_col):
    grid_spec = pltpu.PrefetchScalarGridSpec(
        num_scalar_prefetch=2,
        grid=(NT,),
        in_specs=[
            pl.BlockSpec((TILE, D_MODEL), lambda i, ex, v: (i, 0)),
            pl.BlockSpec((1, D_MODEL, D_FF), lambda i, ex, v: (ex[i], 0, 0)),
            pl.BlockSpec((1, 1, D_FF), lambda i, ex, v: (ex[i], 0, 0)),
            pl.BlockSpec((1, D_FF, D_MODEL), lambda i, ex, v: (ex[i], 0, 0)),
            pl.BlockSpec((1, 1, D_MODEL), lambda i, ex, v: (ex[i], 0, 0)),
            pl.BlockSpec((TILE, 1), lambda i, ex, v: (i, 0)),
        ],
        out_specs=pl.BlockSpec((TILE, D_MODEL), lambda i, ex, v: (i, 0)),
    )
    return pl.pallas_call(
        _ffn_body,
        grid_spec=grid_spec,
        out_shape=jax.ShapeDtypeStruct((P, D_MODEL), jnp.float32),
        compiler_params=pltpu.CompilerParams(
            dimension_semantics=("arbitrary",),
            vmem_limit_bytes=100 * 1024 * 1024),
    )(ex, valid, xs,
      w1, b1.reshape(NUM_EXPERTS, 1, D_FF),
      w2, b2.reshape(NUM_EXPERTS, 1, D_MODEL), ws_col)


_TOK_W = N_TOK // NW


@functools.lru_cache(maxsize=None)
def _build_sc_gather():
    mesh = plsc.VectorSubcoreMesh(core_axis_name="c", subcore_axis_name="s",
                                  num_cores=NC, num_subcores=NS)

    @functools.partial(
        pl.kernel,
        out_type=jax.ShapeDtypeStruct((N_TOK, D_MODEL), jnp.float32),
        mesh=mesh,
        scratch_types=[
            pltpu.VMEM((_TOK_W,), jnp.int32),
            pltpu.VMEM((_TOK_W, D_MODEL), jnp.float32),
            pltpu.SemaphoreType.DMA,
        ],
        compiler_params=pltpu.CompilerParams(needs_layout_passes=False),
    )
    def _sc_gather(ys_hbm, pos_hbm, out_hbm, pos_v, rows_v, sem):
        wid = lax.axis_index("s") * NC + lax.axis_index("c")
        pltpu.sync_copy(pos_hbm.at[pl.ds(wid * _TOK_W, _TOK_W)], pos_v)
        pltpu.async_copy(ys_hbm.at[pos_v], rows_v, sem).wait()
        pltpu.sync_copy(rows_v, out_hbm.at[pl.ds(wid * _TOK_W, _TOK_W)])

    return _sc_gather


def kernel(x, gate_w, gate_b, w1, b1, w2, b2):
    batch, seq_len, hidden = x.shape
    x_flat = x.reshape(N_TOK, D_MODEL)
    pos2, wtok2, ex2, valid2 = _route(x_flat, gate_w, gate_b)
    pos = pos2.reshape(N_TOK)
    wtok = wtok2.reshape(N_TOK)
    xs, ws = _build_sc_scatter()(x_flat, pos, wtok)
    ys = _ffn(ex2.reshape(NT), valid2.reshape(NT), xs,
              w1, b1, w2, b2, ws.reshape(P, 1))
    out = _build_sc_gather()(ys, pos)
    return out.reshape(batch, seq_len, hidden)


# speedup vs baseline: 1.5524x; 1.1134x over previous
import functools

import jax
import jax.numpy as jnp
from jax import lax
from jax.experimental import pallas as pl
from jax.experimental.pallas import tpu as pltpu
from jax.experimental.pallas import tpu_sc as plsc

D_MODEL = 768
NUM_EXPERTS = 16
D_FF = 4 * D_MODEL
N_TOK = 2048
TILE = 128
NT = N_TOK // TILE + NUM_EXPERTS
P = NT * TILE
NC, NS = 2, 16
NW = NC * NS
LANES = 16


def _route_body(x_ref, gw_ref, gb_ref, pos_ref, wtok_ref, ex_ref, valid_ref):
    x = x_ref[...]
    gw = gw_ref[...]
    gb = gb_ref[...]
    logits = jnp.dot(x, gw, preferred_element_type=jnp.float32) + gb
    m = jnp.max(logits, axis=-1, keepdims=True)
    e = jnp.exp(logits - m)
    w = e / jnp.sum(e, axis=-1, keepdims=True)
    wmax = jnp.max(w, axis=-1, keepdims=True)
    eids = lax.broadcasted_iota(jnp.int32, (N_TOK, NUM_EXPERTS), 1)
    expert = jnp.min(jnp.where(w == wmax, eids, NUM_EXPERTS), axis=-1,
                     keepdims=True)
    onehot = (eids == expert).astype(jnp.float32)

    stl_r = lax.broadcasted_iota(jnp.int32, (TILE, TILE), 0)
    stl_c = lax.broadcasted_iota(jnp.int32, (TILE, TILE), 1)
    stl = (stl_c < stl_r).astype(jnp.float32)
    ranks = []
    counts = jnp.zeros((1, NUM_EXPERTS), jnp.float32)
    for c in range(N_TOK // TILE):
        oh_c = onehot[c * TILE:(c + 1) * TILE, :]
        cs = jnp.dot(stl, oh_c, preferred_element_type=jnp.float32) + counts
        ranks.append(jnp.sum(cs * oh_c, axis=-1, keepdims=True))
        counts = counts + jnp.sum(oh_c, axis=0, keepdims=True)
    rank = jnp.concatenate(ranks, axis=0)

    tiles = jnp.ceil(counts / TILE)
    ee_r = lax.broadcasted_iota(jnp.int32, (NUM_EXPERTS, NUM_EXPERTS), 0)
    ee_c = lax.broadcasted_iota(jnp.int32, (NUM_EXPERTS, NUM_EXPERTS), 1)
    excl = jnp.dot(tiles, (ee_r < ee_c).astype(jnp.float32),
                   preferred_element_type=jnp.float32)
    inc = excl + tiles
    pad_start = TILE * excl

    pos = jnp.sum(onehot * pad_start, axis=-1, keepdims=True) + rank
    pos_ref[...] = pos.astype(jnp.int32)
    wtok_ref[...] = wmax

    total = jnp.max(inc)
    it = lax.broadcasted_iota(jnp.int32, (NT, NUM_EXPERTS), 0).astype(jnp.float32)
    ex_full = jnp.sum((jnp.broadcast_to(inc, (NT, NUM_EXPERTS)) <= it)
                      .astype(jnp.int32), axis=-1, keepdims=True)
    ex_last = jnp.sum((inc <= total - 1.0).astype(jnp.int32), axis=-1,
                      keepdims=True)
    tcol = lax.broadcasted_iota(jnp.int32, (NT, 1), 0).astype(jnp.float32)
    valid = tcol < total
    ex = jnp.where(valid, jnp.minimum(ex_full, NUM_EXPERTS - 1),
                   jnp.broadcast_to(ex_last, (NT, 1)))
    ex_ref[...] = ex
    valid_ref[...] = valid.astype(jnp.int32)


def _route(x_flat, gate_w, gate_b):
    return pl.pallas_call(
        _route_body,
        out_shape=[
            jax.ShapeDtypeStruct((N_TOK, 1), jnp.int32),
            jax.ShapeDtypeStruct((N_TOK, 1), jnp.float32),
            jax.ShapeDtypeStruct((NT, 1), jnp.int32),
            jax.ShapeDtypeStruct((NT, 1), jnp.int32),
        ],
    )(x_flat, gate_w, gate_b.reshape(1, NUM_EXPERTS))


@functools.lru_cache(maxsize=None)
def _build_sc_scatter():
    mesh = plsc.VectorSubcoreMesh(core_axis_name="c", subcore_axis_name="s",
                                  num_cores=NC, num_subcores=NS)

    @functools.partial(
        pl.kernel,
        out_type=[
            jax.ShapeDtypeStruct((P, D_MODEL), jnp.float32),
            jax.ShapeDtypeStruct((P, 128), jnp.float32),
        ],
        mesh=mesh,
        scratch_types=[
            pltpu.VMEM((_TOK_W,), jnp.int32),
            pltpu.VMEM((_TOK_W,), jnp.float32),
            pltpu.VMEM((_TOK_W, D_MODEL), jnp.float32),
            pltpu.VMEM((_TOK_W, 128), jnp.float32),
            pltpu.SemaphoreType.DMA,
            pltpu.SemaphoreType.DMA,
            pltpu.SemaphoreType.DMA,
        ],
        compiler_params=pltpu.CompilerParams(needs_layout_passes=False),
    )
    def _sc_scatter(x_hbm, pos_hbm, wtok_hbm, xs_hbm, ws_hbm,
                    pos_v, wtok_v, rows_v, wrows_v, sem_p, sem_r, sem_w):
        wid = lax.axis_index("s") * NC + lax.axis_index("c")
        base = wid * _TOK_W
        cp_p = pltpu.async_copy(pos_hbm.at[pl.ds(base, _TOK_W)], pos_v, sem_p)
        cp_r = pltpu.async_copy(x_hbm.at[pl.ds(base, _TOK_W)], rows_v, sem_r)
        cp_w = pltpu.async_copy(wtok_hbm.at[pl.ds(base, _TOK_W)], wtok_v, sem_w)
        cp_p.wait()
        cp_r.wait()
        s_r = pltpu.async_copy(rows_v, xs_hbm.at[pos_v], sem_r)
        cp_w.wait()

        def wrow_body(c, carry):
            chunk = wtok_v[pl.ds(c * LANES, LANES)]
            for l in range(LANES):
                wrows_v[c * LANES + l, pl.ds(0, LANES)] = jnp.full(
                    (LANES,), chunk[l], jnp.float32)
            return carry

        lax.fori_loop(0, _TOK_W // LANES, wrow_body, 0)
        s_w = pltpu.async_copy(wrows_v, ws_hbm.at[pos_v], sem_w)
        s_w.wait()
        s_r.wait()

    return _sc_scatter


def _ffn_body(ex_ref, valid_ref, xs_ref, w1_ref, b1_ref, w2_ref, b2_ref,
              ws_ref, ys_ref):
    i = pl.program_id(0)

    @pl.when(valid_ref[i] != 0)
    def _():
        xt = xs_ref[...].astype(jnp.bfloat16)
        h = jnp.dot(xt, w1_ref[0].astype(jnp.bfloat16),
                    preferred_element_type=jnp.float32)
        h = h + b1_ref[0]
        h = h * 0.5 * (1.0 + lax.erf(h * 0.7071067811865476))
        y = jnp.dot(h.astype(jnp.bfloat16), w2_ref[0].astype(jnp.bfloat16),
                    preferred_element_type=jnp.float32)
        ys_ref[...] = (y + b2_ref[0]) * ws_ref[...][:, 0:1]


def _ffn(ex, valid, xs, w1, b1, w2, b2, ws_col):
    grid_spec = pltpu.PrefetchScalarGridSpec(
        num_scalar_prefetch=2,
        grid=(NT,),
        in_specs=[
            pl.BlockSpec((TILE, D_MODEL), lambda i, ex, v: (i, 0)),
            pl.BlockSpec((1, D_MODEL, D_FF), lambda i, ex, v: (ex[i], 0, 0)),
            pl.BlockSpec((1, 1, D_FF), lambda i, ex, v: (ex[i], 0, 0)),
            pl.BlockSpec((1, D_FF, D_MODEL), lambda i, ex, v: (ex[i], 0, 0)),
            pl.BlockSpec((1, 1, D_MODEL), lambda i, ex, v: (ex[i], 0, 0)),
            pl.BlockSpec((TILE, 128), lambda i, ex, v: (i, 0)),
        ],
        out_specs=pl.BlockSpec((TILE, D_MODEL), lambda i, ex, v: (i, 0)),
    )
    return pl.pallas_call(
        _ffn_body,
        grid_spec=grid_spec,
        out_shape=jax.ShapeDtypeStruct((P, D_MODEL), jnp.float32),
        compiler_params=pltpu.CompilerParams(
            dimension_semantics=("arbitrary",),
            vmem_limit_bytes=100 * 1024 * 1024),
    )(ex, valid, xs,
      w1, b1.reshape(NUM_EXPERTS, 1, D_FF),
      w2, b2.reshape(NUM_EXPERTS, 1, D_MODEL), ws_col)


_TOK_W = N_TOK // NW


@functools.lru_cache(maxsize=None)
def _build_sc_gather():
    mesh = plsc.VectorSubcoreMesh(core_axis_name="c", subcore_axis_name="s",
                                  num_cores=NC, num_subcores=NS)

    @functools.partial(
        pl.kernel,
        out_type=jax.ShapeDtypeStruct((N_TOK, D_MODEL), jnp.float32),
        mesh=mesh,
        scratch_types=[
            pltpu.VMEM((_TOK_W,), jnp.int32),
            pltpu.VMEM((_TOK_W, D_MODEL), jnp.float32),
            pltpu.SemaphoreType.DMA,
        ],
        compiler_params=pltpu.CompilerParams(needs_layout_passes=False),
    )
    def _sc_gather(ys_hbm, pos_hbm, out_hbm, pos_v, rows_v, sem):
        wid = lax.axis_index("s") * NC + lax.axis_index("c")
        pltpu.sync_copy(pos_hbm.at[pl.ds(wid * _TOK_W, _TOK_W)], pos_v)
        pltpu.async_copy(ys_hbm.at[pos_v], rows_v, sem).wait()
        pltpu.sync_copy(rows_v, out_hbm.at[pl.ds(wid * _TOK_W, _TOK_W)])

    return _sc_gather


def kernel(x, gate_w, gate_b, w1, b1, w2, b2):
    batch, seq_len, hidden = x.shape
    x_flat = x.reshape(N_TOK, D_MODEL)
    pos2, wtok2, ex2, valid2 = _route(x_flat, gate_w, gate_b)
    pos = pos2.reshape(N_TOK)
    wtok = wtok2.reshape(N_TOK)
    xs, ws = _build_sc_scatter()(x_flat, pos, wtok)
    ys = _ffn(ex2.reshape(NT), valid2.reshape(NT), xs,
              w1, b1, w2, b2, ws)
    out = _build_sc_gather()(ys, pos)
    return out.reshape(batch, seq_len, hidden)


# speedup vs baseline: 1.5986x; 1.0298x over previous
import functools

import jax
import jax.numpy as jnp
from jax import lax
from jax.experimental import pallas as pl
from jax.experimental.pallas import tpu as pltpu
from jax.experimental.pallas import tpu_sc as plsc

D_MODEL = 768
NUM_EXPERTS = 16
D_FF = 4 * D_MODEL
N_TOK = 2048
TILE = 128
NT = N_TOK // TILE + NUM_EXPERTS
P = NT * TILE
NC, NS = 2, 16
NW = NC * NS
LANES = 16


def _route_body(x_ref, gw_ref, gb_ref, pos_ref, wtok_ref, ex_ref, valid_ref):
    x = x_ref[...]
    gw = gw_ref[...]
    gb = gb_ref[...]
    logits = lax.dot_general(gw, x, (((0,), (1,)), ((), ())),
                             preferred_element_type=jnp.float32) + gb
    m = jnp.max(logits, axis=0, keepdims=True)
    e = jnp.exp(logits - m)
    w = e / jnp.sum(e, axis=0, keepdims=True)
    wmax = jnp.max(w, axis=0, keepdims=True)
    eids = lax.broadcasted_iota(jnp.int32, (NUM_EXPERTS, N_TOK), 0)
    expert = jnp.min(jnp.where(w == wmax, eids, NUM_EXPERTS), axis=0,
                     keepdims=True)
    onehot = (eids == expert).astype(jnp.float32)

    stu_r = lax.broadcasted_iota(jnp.int32, (TILE, TILE), 0)
    stu_c = lax.broadcasted_iota(jnp.int32, (TILE, TILE), 1)
    stu = (stu_r < stu_c).astype(jnp.float32)
    ranks = []
    counts = jnp.zeros((NUM_EXPERTS, 1), jnp.float32)
    for c in range(N_TOK // TILE):
        oh_c = onehot[:, c * TILE:(c + 1) * TILE]
        cs = jnp.dot(oh_c, stu, preferred_element_type=jnp.float32) + counts
        ranks.append(jnp.sum(cs * oh_c, axis=0, keepdims=True))
        counts = counts + jnp.sum(oh_c, axis=1, keepdims=True)
    rank = jnp.concatenate(ranks, axis=1)

    tiles = jnp.ceil(counts / TILE)
    ee_r = lax.broadcasted_iota(jnp.int32, (NUM_EXPERTS, NUM_EXPERTS), 0)
    ee_c = lax.broadcasted_iota(jnp.int32, (NUM_EXPERTS, NUM_EXPERTS), 1)
    inc = jnp.dot((ee_c <= ee_r).astype(jnp.float32), tiles,
                  preferred_element_type=jnp.float32)
    pad_start = TILE * (inc - tiles)

    pos = jnp.sum(onehot * pad_start, axis=0, keepdims=True) + rank
    pos_ref[...] = pos.astype(jnp.int32)
    wtok_ref[...] = wmax

    total = jnp.max(inc)
    it = lax.broadcasted_iota(jnp.int32, (NUM_EXPERTS, NT), 1).astype(jnp.float32)
    ex_full = jnp.sum((jnp.broadcast_to(inc, (NUM_EXPERTS, NT)) <= it)
                      .astype(jnp.int32), axis=0, keepdims=True)
    ex_last = jnp.sum((inc <= total - 1.0).astype(jnp.int32), axis=0,
                      keepdims=True)
    trow = lax.broadcasted_iota(jnp.int32, (1, NT), 1).astype(jnp.float32)
    valid = trow < total
    ex = jnp.where(valid, jnp.minimum(ex_full, NUM_EXPERTS - 1),
                   jnp.broadcast_to(ex_last, (1, NT)))
    ex_ref[...] = ex
    valid_ref[...] = valid.astype(jnp.int32)


def _route(x_flat, gate_w, gate_b):
    return pl.pallas_call(
        _route_body,
        out_shape=[
            jax.ShapeDtypeStruct((1, N_TOK), jnp.int32),
            jax.ShapeDtypeStruct((1, N_TOK), jnp.float32),
            jax.ShapeDtypeStruct((1, NT), jnp.int32),
            jax.ShapeDtypeStruct((1, NT), jnp.int32),
        ],
    )(x_flat, gate_w, gate_b.reshape(NUM_EXPERTS, 1))


@functools.lru_cache(maxsize=None)
def _build_sc_scatter():
    mesh = plsc.VectorSubcoreMesh(core_axis_name="c", subcore_axis_name="s",
                                  num_cores=NC, num_subcores=NS)

    @functools.partial(
        pl.kernel,
        out_type=[
            jax.ShapeDtypeStruct((P, D_MODEL), jnp.float32),
            jax.ShapeDtypeStruct((P, 128), jnp.float32),
        ],
        mesh=mesh,
        scratch_types=[
            pltpu.VMEM((_TOK_W,), jnp.int32),
            pltpu.VMEM((_TOK_W,), jnp.float32),
            pltpu.VMEM((_TOK_W, D_MODEL), jnp.float32),
            pltpu.VMEM((_TOK_W, 128), jnp.float32),
            pltpu.SemaphoreType.DMA,
            pltpu.SemaphoreType.DMA,
            pltpu.SemaphoreType.DMA,
        ],
        compiler_params=pltpu.CompilerParams(needs_layout_passes=False),
    )
    def _sc_scatter(x_hbm, pos_hbm, wtok_hbm, xs_hbm, ws_hbm,
                    pos_v, wtok_v, rows_v, wrows_v, sem_p, sem_r, sem_w):
        wid = lax.axis_index("s") * NC + lax.axis_index("c")
        base = wid * _TOK_W
        cp_p = pltpu.async_copy(pos_hbm.at[0, pl.ds(base, _TOK_W)], pos_v, sem_p)
        cp_r = pltpu.async_copy(x_hbm.at[pl.ds(base, _TOK_W)], rows_v, sem_r)
        cp_w = pltpu.async_copy(wtok_hbm.at[0, pl.ds(base, _TOK_W)], wtok_v, sem_w)
        cp_p.wait()
        cp_r.wait()
        s_r = pltpu.async_copy(rows_v, xs_hbm.at[pos_v], sem_r)
        cp_w.wait()

        def wrow_body(c, carry):
            chunk = wtok_v[pl.ds(c * LANES, LANES)]
            for l in range(LANES):
                wrows_v[c * LANES + l, pl.ds(0, LANES)] = jnp.full(
                    (LANES,), chunk[l], jnp.float32)
            return carry

        lax.fori_loop(0, _TOK_W // LANES, wrow_body, 0)
        s_w = pltpu.async_copy(wrows_v, ws_hbm.at[pos_v], sem_w)
        s_w.wait()
        s_r.wait()

    return _sc_scatter


def _ffn_body(ex_ref, valid_ref, xs_ref, w1_ref, b1_ref, w2_ref, b2_ref,
              ws_ref, ys_ref):
    i = pl.program_id(0)

    @pl.when(valid_ref[0, i] != 0)
    def _():
        xt = xs_ref[...].astype(jnp.bfloat16)
        h = jnp.dot(xt, w1_ref[0].astype(jnp.bfloat16),
                    preferred_element_type=jnp.float32)
        h = h + b1_ref[0]
        h = h * 0.5 * (1.0 + lax.erf(h * 0.7071067811865476))
        y = jnp.dot(h.astype(jnp.bfloat16), w2_ref[0].astype(jnp.bfloat16),
                    preferred_element_type=jnp.float32)
        ys_ref[...] = (y + b2_ref[0]) * ws_ref[...][:, 0:1]


def _ffn(ex, valid, xs, w1, b1, w2, b2, ws_col):
    grid_spec = pltpu.PrefetchScalarGridSpec(
        num_scalar_prefetch=2,
        grid=(NT,),
        in_specs=[
            pl.BlockSpec((TILE, D_MODEL), lambda i, ex, v: (i, 0)),
            pl.BlockSpec((1, D_MODEL, D_FF), lambda i, ex, v: (ex[0, i], 0, 0)),
            pl.BlockSpec((1, 1, D_FF), lambda i, ex, v: (ex[0, i], 0, 0)),
            pl.BlockSpec((1, D_FF, D_MODEL), lambda i, ex, v: (ex[0, i], 0, 0)),
            pl.BlockSpec((1, 1, D_MODEL), lambda i, ex, v: (ex[0, i], 0, 0)),
            pl.BlockSpec((TILE, 128), lambda i, ex, v: (i, 0)),
        ],
        out_specs=pl.BlockSpec((TILE, D_MODEL), lambda i, ex, v: (i, 0)),
    )
    return pl.pallas_call(
        _ffn_body,
        grid_spec=grid_spec,
        out_shape=jax.ShapeDtypeStruct((P, D_MODEL), jnp.float32),
        compiler_params=pltpu.CompilerParams(
            dimension_semantics=("arbitrary",),
            vmem_limit_bytes=100 * 1024 * 1024),
    )(ex, valid, xs,
      w1, b1.reshape(NUM_EXPERTS, 1, D_FF),
      w2, b2.reshape(NUM_EXPERTS, 1, D_MODEL), ws_col)


_TOK_W = N_TOK // NW


@functools.lru_cache(maxsize=None)
def _build_sc_gather():
    mesh = plsc.VectorSubcoreMesh(core_axis_name="c", subcore_axis_name="s",
                                  num_cores=NC, num_subcores=NS)

    @functools.partial(
        pl.kernel,
        out_type=jax.ShapeDtypeStruct((N_TOK, D_MODEL), jnp.float32),
        mesh=mesh,
        scratch_types=[
            pltpu.VMEM((_TOK_W,), jnp.int32),
            pltpu.VMEM((_TOK_W, D_MODEL), jnp.float32),
            pltpu.SemaphoreType.DMA,
        ],
        compiler_params=pltpu.CompilerParams(needs_layout_passes=False),
    )
    def _sc_gather(ys_hbm, pos_hbm, out_hbm, pos_v, rows_v, sem):
        wid = lax.axis_index("s") * NC + lax.axis_index("c")
        pltpu.sync_copy(pos_hbm.at[0, pl.ds(wid * _TOK_W, _TOK_W)], pos_v)
        pltpu.async_copy(ys_hbm.at[pos_v], rows_v, sem).wait()
        pltpu.sync_copy(rows_v, out_hbm.at[pl.ds(wid * _TOK_W, _TOK_W)])

    return _sc_gather


def kernel(x, gate_w, gate_b, w1, b1, w2, b2):
    batch, seq_len, hidden = x.shape
    x_flat = x.reshape(N_TOK, D_MODEL)
    pos, wtok, ex, valid = _route(x_flat, gate_w, gate_b)
    xs, ws = _build_sc_scatter()(x_flat, pos, wtok)
    ys = _ffn(ex, valid, xs, w1, b1, w2, b2, ws)
    out = _build_sc_gather()(ys, pos)
    return out.reshape(batch, seq_len, hidden)


# speedup vs baseline: 1.6011x; 1.0015x over previous
import functools

import jax
import jax.numpy as jnp
from jax import lax
from jax.experimental import pallas as pl
from jax.experimental.pallas import tpu as pltpu
from jax.experimental.pallas import tpu_sc as plsc

D_MODEL = 768
NUM_EXPERTS = 16
D_FF = 4 * D_MODEL
N_TOK = 2048
TILE = 128
NT = N_TOK // TILE + NUM_EXPERTS
P = NT * TILE
NC, NS = 2, 16
NW = NC * NS
LANES = 16
HF = D_FF // 2


def _route_body(x_ref, gw_ref, gb_ref, pos_ref, wtok_ref, ex_ref, valid_ref):
    x = x_ref[...]
    gw = gw_ref[...]
    gb = gb_ref[...]
    logits = lax.dot_general(gw, x, (((0,), (1,)), ((), ())),
                             preferred_element_type=jnp.float32) + gb
    m = jnp.max(logits, axis=0, keepdims=True)
    e = jnp.exp(logits - m)
    w = e / jnp.sum(e, axis=0, keepdims=True)
    wmax = jnp.max(w, axis=0, keepdims=True)
    eids = lax.broadcasted_iota(jnp.int32, (NUM_EXPERTS, N_TOK), 0)
    expert = jnp.min(jnp.where(w == wmax, eids, NUM_EXPERTS), axis=0,
                     keepdims=True)
    onehot = (eids == expert).astype(jnp.float32)

    stu_r = lax.broadcasted_iota(jnp.int32, (TILE, TILE), 0)
    stu_c = lax.broadcasted_iota(jnp.int32, (TILE, TILE), 1)
    stu = (stu_r < stu_c).astype(jnp.float32)
    ranks = []
    counts = jnp.zeros((NUM_EXPERTS, 1), jnp.float32)
    for c in range(N_TOK // TILE):
        oh_c = onehot[:, c * TILE:(c + 1) * TILE]
        cs = jnp.dot(oh_c, stu, preferred_element_type=jnp.float32) + counts
        ranks.append(jnp.sum(cs * oh_c, axis=0, keepdims=True))
        counts = counts + jnp.sum(oh_c, axis=1, keepdims=True)
    rank = jnp.concatenate(ranks, axis=1)

    tiles = jnp.ceil(counts / TILE)
    ee_r = lax.broadcasted_iota(jnp.int32, (NUM_EXPERTS, NUM_EXPERTS), 0)
    ee_c = lax.broadcasted_iota(jnp.int32, (NUM_EXPERTS, NUM_EXPERTS), 1)
    inc = jnp.dot((ee_c <= ee_r).astype(jnp.float32), tiles,
                  preferred_element_type=jnp.float32)
    pad_start = TILE * (inc - tiles)

    pos = jnp.sum(onehot * pad_start, axis=0, keepdims=True) + rank
    pos_ref[...] = pos.astype(jnp.int32)
    wtok_ref[...] = wmax

    total = jnp.max(inc)
    it = lax.broadcasted_iota(jnp.int32, (NUM_EXPERTS, NT), 1).astype(jnp.float32)
    ex_full = jnp.sum((jnp.broadcast_to(inc, (NUM_EXPERTS, NT)) <= it)
                      .astype(jnp.int32), axis=0, keepdims=True)
    ex_last = jnp.sum((inc <= total - 1.0).astype(jnp.int32), axis=0,
                      keepdims=True)
    trow = lax.broadcasted_iota(jnp.int32, (1, NT), 1).astype(jnp.float32)
    valid = trow < total
    ex = jnp.where(valid, jnp.minimum(ex_full, NUM_EXPERTS - 1),
                   jnp.broadcast_to(ex_last, (1, NT)))
    ex_ref[...] = ex
    valid_ref[...] = valid.astype(jnp.int32)


def _route(x_flat, gate_w, gate_b):
    return pl.pallas_call(
        _route_body,
        out_shape=[
            jax.ShapeDtypeStruct((1, N_TOK), jnp.int32),
            jax.ShapeDtypeStruct((1, N_TOK), jnp.float32),
            jax.ShapeDtypeStruct((1, NT), jnp.int32),
            jax.ShapeDtypeStruct((1, NT), jnp.int32),
        ],
    )(x_flat, gate_w, gate_b.reshape(NUM_EXPERTS, 1))


@functools.lru_cache(maxsize=None)
def _build_sc_scatter():
    mesh = plsc.VectorSubcoreMesh(core_axis_name="c", subcore_axis_name="s",
                                  num_cores=NC, num_subcores=NS)

    @functools.partial(
        pl.kernel,
        out_type=[
            jax.ShapeDtypeStruct((P, D_MODEL), jnp.float32),
            jax.ShapeDtypeStruct((P, 128), jnp.float32),
        ],
        mesh=mesh,
        scratch_types=[
            pltpu.VMEM((_TOK_W,), jnp.int32),
            pltpu.VMEM((_TOK_W,), jnp.float32),
            pltpu.VMEM((_TOK_W, D_MODEL), jnp.float32),
            pltpu.VMEM((_TOK_W, 128), jnp.float32),
            pltpu.SemaphoreType.DMA,
            pltpu.SemaphoreType.DMA,
            pltpu.SemaphoreType.DMA,
        ],
        compiler_params=pltpu.CompilerParams(needs_layout_passes=False),
    )
    def _sc_scatter(x_hbm, pos_hbm, wtok_hbm, xs_hbm, ws_hbm,
                    pos_v, wtok_v, rows_v, wrows_v, sem_p, sem_r, sem_w):
        wid = lax.axis_index("s") * NC + lax.axis_index("c")
        base = wid * _TOK_W
        cp_p = pltpu.async_copy(pos_hbm.at[0, pl.ds(base, _TOK_W)], pos_v, sem_p)
        cp_r = pltpu.async_copy(x_hbm.at[pl.ds(base, _TOK_W)], rows_v, sem_r)
        cp_w = pltpu.async_copy(wtok_hbm.at[0, pl.ds(base, _TOK_W)], wtok_v, sem_w)
        cp_p.wait()
        cp_r.wait()
        s_r = pltpu.async_copy(rows_v, xs_hbm.at[pos_v], sem_r)
        cp_w.wait()

        def wrow_body(c, carry):
            chunk = wtok_v[pl.ds(c * LANES, LANES)]
            for l in range(LANES):
                wrows_v[c * LANES + l, pl.ds(0, LANES)] = jnp.full(
                    (LANES,), chunk[l], jnp.float32)
            return carry

        lax.fori_loop(0, _TOK_W // LANES, wrow_body, 0)
        s_w = pltpu.async_copy(wrows_v, ws_hbm.at[pos_v], sem_w)
        s_w.wait()
        s_r.wait()

    return _sc_scatter


def _ffn_body(ex_ref, valid_ref, xs_ref, w1a_ref, w1b_ref, b1_ref,
              w2a_ref, w2b_ref, b2_ref, ws_ref, ys_ref):
    i = pl.program_id(0)

    @pl.when(valid_ref[0, i] != 0)
    def _():
        xt = xs_ref[...].astype(jnp.bfloat16)
        ha = jnp.dot(xt, w1a_ref[0].astype(jnp.bfloat16),
                     preferred_element_type=jnp.float32) + b1_ref[0, :, :HF]
        hb = jnp.dot(xt, w1b_ref[0].astype(jnp.bfloat16),
                     preferred_element_type=jnp.float32) + b1_ref[0, :, HF:]
        ha = ha * 0.5 * (1.0 + lax.erf(ha * 0.7071067811865476))
        hb = hb * 0.5 * (1.0 + lax.erf(hb * 0.7071067811865476))
        y = jnp.dot(ha.astype(jnp.bfloat16), w2a_ref[0].astype(jnp.bfloat16),
                    preferred_element_type=jnp.float32)
        y = y + jnp.dot(hb.astype(jnp.bfloat16), w2b_ref[0].astype(jnp.bfloat16),
                        preferred_element_type=jnp.float32)
        ys_ref[...] = (y + b2_ref[0]) * ws_ref[...][:, 0:1]


def _ffn(ex, valid, xs, w1, b1, w2, b2, ws_col):
    grid_spec = pltpu.PrefetchScalarGridSpec(
        num_scalar_prefetch=2,
        grid=(NT,),
        in_specs=[
            pl.BlockSpec((TILE, D_MODEL), lambda i, ex, v: (i, 0)),
            pl.BlockSpec((1, D_MODEL, HF), lambda i, ex, v: (ex[0, i], 0, 0)),
            pl.BlockSpec((1, D_MODEL, HF), lambda i, ex, v: (ex[0, i], 0, 1)),
            pl.BlockSpec((1, 1, D_FF), lambda i, ex, v: (ex[0, i], 0, 0)),
            pl.BlockSpec((1, HF, D_MODEL), lambda i, ex, v: (ex[0, i], 0, 0)),
            pl.BlockSpec((1, HF, D_MODEL), lambda i, ex, v: (ex[0, i], 1, 0)),
            pl.BlockSpec((1, 1, D_MODEL), lambda i, ex, v: (ex[0, i], 0, 0)),
            pl.BlockSpec((TILE, 128), lambda i, ex, v: (i, 0)),
        ],
        out_specs=pl.BlockSpec((TILE, D_MODEL), lambda i, ex, v: (i, 0)),
    )
    return pl.pallas_call(
        _ffn_body,
        grid_spec=grid_spec,
        out_shape=jax.ShapeDtypeStruct((P, D_MODEL), jnp.float32),
        compiler_params=pltpu.CompilerParams(
            dimension_semantics=("arbitrary",),
            vmem_limit_bytes=100 * 1024 * 1024),
    )(ex, valid, xs,
      w1, w1, b1.reshape(NUM_EXPERTS, 1, D_FF),
      w2, w2, b2.reshape(NUM_EXPERTS, 1, D_MODEL), ws_col)


_TOK_W = N_TOK // NW


@functools.lru_cache(maxsize=None)
def _build_sc_gather():
    mesh = plsc.VectorSubcoreMesh(core_axis_name="c", subcore_axis_name="s",
                                  num_cores=NC, num_subcores=NS)

    @functools.partial(
        pl.kernel,
        out_type=jax.ShapeDtypeStruct((N_TOK, D_MODEL), jnp.float32),
        mesh=mesh,
        scratch_types=[
            pltpu.VMEM((_TOK_W,), jnp.int32),
            pltpu.VMEM((_TOK_W, D_MODEL), jnp.float32),
            pltpu.SemaphoreType.DMA,
        ],
        compiler_params=pltpu.CompilerParams(needs_layout_passes=False),
    )
    def _sc_gather(ys_hbm, pos_hbm, out_hbm, pos_v, rows_v, sem):
        wid = lax.axis_index("s") * NC + lax.axis_index("c")
        pltpu.sync_copy(pos_hbm.at[0, pl.ds(wid * _TOK_W, _TOK_W)], pos_v)
        pltpu.async_copy(ys_hbm.at[pos_v], rows_v, sem).wait()
        pltpu.sync_copy(rows_v, out_hbm.at[pl.ds(wid * _TOK_W, _TOK_W)])

    return _sc_gather


def kernel(x, gate_w, gate_b, w1, b1, w2, b2):
    batch, seq_len, hidden = x.shape
    x_flat = x.reshape(N_TOK, D_MODEL)
    pos, wtok, ex, valid = _route(x_flat, gate_w, gate_b)
    xs, ws = _build_sc_scatter()(x_flat, pos, wtok)
    ys = _ffn(ex, valid, xs, w1, b1, w2, b2, ws)
    out = _build_sc_gather()(ys, pos)
    return out.reshape(batch, seq_len, hidden)


# speedup vs baseline: 1.6035x; 1.0015x over previous
import functools

import jax
import jax.numpy as jnp
from jax import lax
from jax.experimental import pallas as pl
from jax.experimental.pallas import tpu as pltpu
from jax.experimental.pallas import tpu_sc as plsc

D_MODEL = 768
NUM_EXPERTS = 16
D_FF = 4 * D_MODEL
N_TOK = 2048
TILE = 128
NT = N_TOK // TILE + NUM_EXPERTS
P = NT * TILE
NC, NS = 2, 16
NW = NC * NS
LANES = 16


def _route_body(x_ref, gw_ref, gb_ref, pos_ref, wtok_ref, ex_ref, valid_ref):
    x = x_ref[...]
    gw = gw_ref[...]
    gb = gb_ref[...]
    logits = lax.dot_general(gw, x, (((0,), (1,)), ((), ())),
                             preferred_element_type=jnp.float32) + gb
    m = jnp.max(logits, axis=0, keepdims=True)
    e = jnp.exp(logits - m)
    w = e / jnp.sum(e, axis=0, keepdims=True)
    wmax = jnp.max(w, axis=0, keepdims=True)
    eids = lax.broadcasted_iota(jnp.int32, (NUM_EXPERTS, N_TOK), 0)
    expert = jnp.min(jnp.where(w == wmax, eids, NUM_EXPERTS), axis=0,
                     keepdims=True)
    onehot = (eids == expert).astype(jnp.float32)

    stu_r = lax.broadcasted_iota(jnp.int32, (TILE, TILE), 0)
    stu_c = lax.broadcasted_iota(jnp.int32, (TILE, TILE), 1)
    stu = (stu_r < stu_c).astype(jnp.float32)
    ranks = []
    counts = jnp.zeros((NUM_EXPERTS, 1), jnp.float32)
    for c in range(N_TOK // TILE):
        oh_c = onehot[:, c * TILE:(c + 1) * TILE]
        cs = jnp.dot(oh_c, stu, preferred_element_type=jnp.float32) + counts
        ranks.append(jnp.sum(cs * oh_c, axis=0, keepdims=True))
        counts = counts + jnp.sum(oh_c, axis=1, keepdims=True)
    rank = jnp.concatenate(ranks, axis=1)

    tiles = jnp.ceil(counts / TILE)
    ee_r = lax.broadcasted_iota(jnp.int32, (NUM_EXPERTS, NUM_EXPERTS), 0)
    ee_c = lax.broadcasted_iota(jnp.int32, (NUM_EXPERTS, NUM_EXPERTS), 1)
    inc = jnp.dot((ee_c <= ee_r).astype(jnp.float32), tiles,
                  preferred_element_type=jnp.float32)
    pad_start = TILE * (inc - tiles)

    pos = jnp.sum(onehot * pad_start, axis=0, keepdims=True) + rank
    pos_ref[...] = pos.astype(jnp.int32)
    wtok_ref[...] = wmax

    total = jnp.max(inc)
    it = lax.broadcasted_iota(jnp.int32, (NUM_EXPERTS, NT), 1).astype(jnp.float32)
    ex_full = jnp.sum((jnp.broadcast_to(inc, (NUM_EXPERTS, NT)) <= it)
                      .astype(jnp.int32), axis=0, keepdims=True)
    ex_last = jnp.sum((inc <= total - 1.0).astype(jnp.int32), axis=0,
                      keepdims=True)
    trow = lax.broadcasted_iota(jnp.int32, (1, NT), 1).astype(jnp.float32)
    valid = trow < total
    ex = jnp.where(valid, jnp.minimum(ex_full, NUM_EXPERTS - 1),
                   jnp.broadcast_to(ex_last, (1, NT)))
    ex_ref[...] = ex
    valid_ref[...] = valid.astype(jnp.int32)


def _route(x_flat, gate_w, gate_b):
    return pl.pallas_call(
        _route_body,
        out_shape=[
            jax.ShapeDtypeStruct((1, N_TOK), jnp.int32),
            jax.ShapeDtypeStruct((1, N_TOK), jnp.float32),
            jax.ShapeDtypeStruct((1, NT), jnp.int32),
            jax.ShapeDtypeStruct((1, NT), jnp.int32),
        ],
    )(x_flat, gate_w, gate_b.reshape(NUM_EXPERTS, 1))


@functools.lru_cache(maxsize=None)
def _build_sc_scatter():
    mesh = plsc.VectorSubcoreMesh(core_axis_name="c", subcore_axis_name="s",
                                  num_cores=NC, num_subcores=NS)

    @functools.partial(
        pl.kernel,
        out_type=[
            jax.ShapeDtypeStruct((P, D_MODEL), jnp.float32),
            jax.ShapeDtypeStruct((P, 128), jnp.float32),
        ],
        mesh=mesh,
        scratch_types=[
            pltpu.VMEM((_TOK_W,), jnp.int32),
            pltpu.VMEM((_TOK_W,), jnp.float32),
            pltpu.VMEM((_TOK_W, D_MODEL), jnp.float32),
            pltpu.VMEM((_TOK_W, 128), jnp.float32),
            pltpu.SemaphoreType.DMA,
            pltpu.SemaphoreType.DMA,
            pltpu.SemaphoreType.DMA,
        ],
        compiler_params=pltpu.CompilerParams(needs_layout_passes=False),
    )
    def _sc_scatter(x_hbm, pos_hbm, wtok_hbm, xs_hbm, ws_hbm,
                    pos_v, wtok_v, rows_v, wrows_v, sem_p, sem_r, sem_w):
        wid = lax.axis_index("s") * NC + lax.axis_index("c")
        base = wid * _TOK_W
        cp_p = pltpu.async_copy(pos_hbm.at[0, pl.ds(base, _TOK_W)], pos_v, sem_p)
        cp_r = pltpu.async_copy(x_hbm.at[pl.ds(base, _TOK_W)], rows_v, sem_r)
        cp_w = pltpu.async_copy(wtok_hbm.at[0, pl.ds(base, _TOK_W)], wtok_v, sem_w)
        cp_p.wait()
        cp_r.wait()
        s_r = pltpu.async_copy(rows_v, xs_hbm.at[pos_v], sem_r)
        cp_w.wait()

        def wrow_body(c, carry):
            chunk = wtok_v[pl.ds(c * LANES, LANES)]
            for l in range(LANES):
                wrows_v[c * LANES + l, pl.ds(0, LANES)] = jnp.full(
                    (LANES,), chunk[l], jnp.float32)
            return carry

        lax.fori_loop(0, _TOK_W // LANES, wrow_body, 0)
        s_w = pltpu.async_copy(wrows_v, ws_hbm.at[pos_v], sem_w)
        s_w.wait()
        s_r.wait()

    return _sc_scatter


def _ffn_body(ex_ref, valid_ref, xs_ref, w1_ref, b1_ref, w2_ref, b2_ref,
              ws_ref, ys_ref):
    i = pl.program_id(0)

    @pl.when(valid_ref[0, i] != 0)
    def _():
        xt = xs_ref[...].astype(jnp.bfloat16)
        h = jnp.dot(xt, w1_ref[0].astype(jnp.bfloat16),
                    preferred_element_type=jnp.float32)
        h = h + b1_ref[0]
        h = h * 0.5 * (1.0 + lax.erf(h * 0.7071067811865476))
        y = jnp.dot(h.astype(jnp.bfloat16), w2_ref[0].astype(jnp.bfloat16),
                    preferred_element_type=jnp.float32)
        ys_ref[...] = (y + b2_ref[0]) * ws_ref[...][:, 0:1]


def _ffn(ex, valid, xs, w1, b1, w2, b2, ws_col):
    grid_spec = pltpu.PrefetchScalarGridSpec(
        num_scalar_prefetch=2,
        grid=(NT,),
        in_specs=[
            pl.BlockSpec((TILE, D_MODEL), lambda i, ex, v: (i, 0)),
            pl.BlockSpec((1, D_MODEL, D_FF), lambda i, ex, v: (ex[0, i], 0, 0)),
            pl.BlockSpec((1, 1, D_FF), lambda i, ex, v: (ex[0, i], 0, 0)),
            pl.BlockSpec((1, D_FF, D_MODEL), lambda i, ex, v: (ex[0, i], 0, 0)),
            pl.BlockSpec((1, 1, D_MODEL), lambda i, ex, v: (ex[0, i], 0, 0)),
            pl.BlockSpec((TILE, 128), lambda i, ex, v: (i, 0)),
        ],
        out_specs=pl.BlockSpec((TILE, D_MODEL), lambda i, ex, v: (i, 0)),
    )
    return pl.pallas_call(
        _ffn_body,
        grid_spec=grid_spec,
        out_shape=jax.ShapeDtypeStruct((P, D_MODEL), jnp.float32),
        compiler_params=pltpu.CompilerParams(
            dimension_semantics=("arbitrary",),
            vmem_limit_bytes=100 * 1024 * 1024),
    )(ex, valid, xs,
      w1, b1.reshape(NUM_EXPERTS, 1, D_FF),
      w2, b2.reshape(NUM_EXPERTS, 1, D_MODEL), ws_col)


_TOK_W = N_TOK // NW


@functools.lru_cache(maxsize=None)
def _build_sc_gather():
    mesh = plsc.VectorSubcoreMesh(core_axis_name="c", subcore_axis_name="s",
                                  num_cores=NC, num_subcores=NS)

    @functools.partial(
        pl.kernel,
        out_type=jax.ShapeDtypeStruct((N_TOK, D_MODEL), jnp.float32),
        mesh=mesh,
        scratch_types=[
            pltpu.VMEM((_TOK_W,), jnp.int32),
            pltpu.VMEM((_TOK_W, D_MODEL), jnp.float32),
            pltpu.SemaphoreType.DMA,
        ],
        compiler_params=pltpu.CompilerParams(needs_layout_passes=False),
    )
    def _sc_gather(ys_hbm, pos_hbm, out_hbm, pos_v, rows_v, sem):
        wid = lax.axis_index("s") * NC + lax.axis_index("c")
        pltpu.sync_copy(pos_hbm.at[0, pl.ds(wid * _TOK_W, _TOK_W)], pos_v)
        pltpu.async_copy(ys_hbm.at[pos_v], rows_v, sem).wait()
        pltpu.sync_copy(rows_v, out_hbm.at[pl.ds(wid * _TOK_W, _TOK_W)])

    return _sc_gather


def kernel(x, gate_w, gate_b, w1, b1, w2, b2):
    batch, seq_len, hidden = x.shape
    x_flat = x.reshape(N_TOK, D_MODEL)
    pos, wtok, ex, valid = _route(x_flat, gate_w, gate_b)
    xs, ws = _build_sc_scatter()(x_flat, pos, wtok)
    ys = _ffn(ex, valid, xs, w1, b1, w2, b2, ws)
    out = _build_sc_gather()(ys, pos)
    return out.reshape(batch, seq_len, hidden)
